# baseline (speedup 1.0000x reference)
import jax
import jax.numpy as jnp
from jax.experimental import pallas as pl
from jax.experimental.pallas import tpu as pltpu

NUM_ENTS = 1000
NUM_RELS = 24
HIDDEN = 200
B = 8192
BLK = 4096
NB = B // BLK
PADW = 256
WCOMB = PADW + HIDDEN
LOG_HALF = -0.6931471805599453


def _fused_kernel(sub_ref, rel_ref, ent_ref, rele_ref, tim_ref,
                  W_ref, b_ref, x_ref, out_ref):
    sub_row = sub_ref[0].astype(jnp.int16)
    rel_row = rel_ref[0].astype(jnp.int16)

    one = jnp.ones((), jnp.bfloat16)
    zero = jnp.zeros((), jnp.bfloat16)

    ent_iota = jax.lax.broadcasted_iota(jnp.int16, (NUM_ENTS, BLK), 0)
    sub_oh = jnp.where(ent_iota == sub_row, one, zero)
    sub_e = jax.lax.dot_general(
        sub_oh, ent_ref[...].astype(jnp.bfloat16),
        dimension_numbers=(((0,), (0,)), ((), ())),
        preferred_element_type=jnp.float32)

    rel_iota = jax.lax.broadcasted_iota(jnp.int16, (NUM_RELS, BLK), 0)
    rel_oh = jnp.where(rel_iota == rel_row, one, zero)
    rel_e = jax.lax.dot_general(
        rel_oh, rele_ref[...].astype(jnp.bfloat16),
        dimension_numbers=(((0,), (0,)), ((), ())),
        preferred_element_type=jnp.float32)

    tim = tim_ref[...]
    x_ref[:, pl.ds(0, HIDDEN)] = sub_e
    x_ref[:, pl.ds(HIDDEN, HIDDEN)] = rel_e
    x_ref[:, pl.ds(2 * HIDDEN, HIDDEN)] = jnp.broadcast_to(tim, (BLK, HIDDEN))

    W = W_ref[...].astype(jnp.bfloat16)
    scores = (
        jax.lax.dot_general(sub_e.astype(jnp.bfloat16), W[:HIDDEN],
                            dimension_numbers=(((1,), (0,)), ((), ())),
                            preferred_element_type=jnp.float32)
        + jax.lax.dot_general(rel_e.astype(jnp.bfloat16),
                              W[HIDDEN:2 * HIDDEN],
                              dimension_numbers=(((1,), (0,)), ((), ())),
                              preferred_element_type=jnp.float32)
        + jnp.dot(tim.astype(jnp.bfloat16), W[2 * HIDDEN:],
                  preferred_element_type=jnp.float32)
        + b_ref[...])

    gscore = scores[:, :HIDDEN]
    z = jnp.tanh(scores[:, PADW:PADW + HIDDEN])
    ez = jnp.exp(z)
    cscore = ez / jnp.sum(ez, axis=1, keepdims=True)
    out_ref[...] = jnp.log(gscore + cscore) + jnp.float32(LOG_HALF)


def kernel(hist_sub, hist_rel, hist_obj, sub, rel, obj,
           ent_emb, rel_emb, tim_emb, gW, gb, cW, cb):
    del hist_sub, hist_rel, hist_obj, obj
    sub3 = sub.astype(jnp.int32).reshape(NB, 1, BLK)
    rel3 = rel.astype(jnp.int32).reshape(NB, 1, BLK)
    zpad_w = jnp.zeros((3 * HIDDEN, PADW - HIDDEN), jnp.float32)
    W_comb = jnp.concatenate([gW, zpad_w, cW], axis=1)
    b_comb = jnp.concatenate(
        [gb, jnp.zeros((PADW - HIDDEN,), jnp.float32), cb]).reshape(1, WCOMB)

    x_out, log_out = pl.pallas_call(
        _fused_kernel,
        grid=(NB,),
        in_specs=[
            pl.BlockSpec((1, 1, BLK), lambda i: (i, 0, 0)),
            pl.BlockSpec((1, 1, BLK), lambda i: (i, 0, 0)),
            pl.BlockSpec((NUM_ENTS, HIDDEN), lambda i: (0, 0)),
            pl.BlockSpec((NUM_RELS, HIDDEN), lambda i: (0, 0)),
            pl.BlockSpec((1, HIDDEN), lambda i: (0, 0)),
            pl.BlockSpec((3 * HIDDEN, WCOMB), lambda i: (0, 0)),
            pl.BlockSpec((1, WCOMB), lambda i: (0, 0)),
        ],
        out_specs=[
            pl.BlockSpec((BLK, 3 * HIDDEN), lambda i: (i, 0)),
            pl.BlockSpec((BLK, HIDDEN), lambda i: (i, 0)),
        ],
        out_shape=[
            jax.ShapeDtypeStruct((B, 3 * HIDDEN), jnp.float32),
            jax.ShapeDtypeStruct((B, HIDDEN), jnp.float32),
        ],
        compiler_params=pltpu.CompilerParams(
            dimension_semantics=("parallel",)),
    )(sub3, rel3, ent_emb, rel_emb, tim_emb, W_comb, b_comb)
    return log_out, x_out


# speedup vs baseline: 1.1106x; 1.1106x over previous
import jax
import jax.numpy as jnp
from jax.experimental import pallas as pl
from jax.experimental.pallas import tpu as pltpu

NUM_ENTS = 1000
NUM_RELS = 24
HIDDEN = 200
B = 8192
BLK = 2048
HALF = BLK // 2
NB = B // BLK
PADW = 256
WCOMB = PADW + HIDDEN
LOG_HALF = -0.6931471805599453


def _fused_kernel(sub_ref, rel_ref, ent_ref, rele_ref, tim_ref,
                  W_ref, b_ref, x_ref, out_ref):
    sub_row_full = sub_ref[0].astype(jnp.int16)
    rel_row_full = rel_ref[0].astype(jnp.int16)

    one = jnp.ones((), jnp.bfloat16)
    zero = jnp.zeros((), jnp.bfloat16)
    ent_b = ent_ref[...].astype(jnp.bfloat16)
    rele_b = rele_ref[...].astype(jnp.bfloat16)
    tim = tim_ref[...]
    W = W_ref[...].astype(jnp.bfloat16)
    tim_score = (jnp.dot(tim.astype(jnp.bfloat16), W[2 * HIDDEN:],
                         preferred_element_type=jnp.float32)
                 + b_ref[...])
    ent_iota = jax.lax.broadcasted_iota(jnp.int16, (NUM_ENTS, HALF), 0)
    rel_iota = jax.lax.broadcasted_iota(jnp.int16, (NUM_RELS, HALF), 0)

    for h in range(2):
        lo = h * HALF
        sub_row = sub_row_full[:, lo:lo + HALF]
        rel_row = rel_row_full[:, lo:lo + HALF]

        sub_oh = jnp.where(ent_iota == sub_row, one, zero)
        sub_e = jax.lax.dot_general(
            sub_oh, ent_b,
            dimension_numbers=(((0,), (0,)), ((), ())),
            preferred_element_type=jnp.float32)

        rel_oh = jnp.where(rel_iota == rel_row, one, zero)
        rel_e = jax.lax.dot_general(
            rel_oh, rele_b,
            dimension_numbers=(((0,), (0,)), ((), ())),
            preferred_element_type=jnp.float32)

        rows = pl.ds(lo, HALF)
        x_ref[rows, pl.ds(0, HIDDEN)] = sub_e
        x_ref[rows, pl.ds(HIDDEN, HIDDEN)] = rel_e
        x_ref[rows, pl.ds(2 * HIDDEN, HIDDEN)] = jnp.broadcast_to(
            tim, (HALF, HIDDEN))

        scores = (
            jax.lax.dot_general(sub_e.astype(jnp.bfloat16), W[:HIDDEN],
                                dimension_numbers=(((1,), (0,)), ((), ())),
                                preferred_element_type=jnp.float32)
            + jax.lax.dot_general(rel_e.astype(jnp.bfloat16),
                                  W[HIDDEN:2 * HIDDEN],
                                  dimension_numbers=(((1,), (0,)), ((), ())),
                                  preferred_element_type=jnp.float32)
            + tim_score)

        gscore = scores[:, :HIDDEN]
        z = jnp.tanh(scores[:, PADW:PADW + HIDDEN])
        ez = jnp.exp(z)
        cscore = ez / jnp.sum(ez, axis=1, keepdims=True)
        out_ref[rows, :] = jnp.log(gscore + cscore) + jnp.float32(LOG_HALF)


def kernel(hist_sub, hist_rel, hist_obj, sub, rel, obj,
           ent_emb, rel_emb, tim_emb, gW, gb, cW, cb):
    del hist_sub, hist_rel, hist_obj, obj
    sub3 = sub.astype(jnp.int32).reshape(NB, 1, BLK)
    rel3 = rel.astype(jnp.int32).reshape(NB, 1, BLK)
    zpad_w = jnp.zeros((3 * HIDDEN, PADW - HIDDEN), jnp.float32)
    W_comb = jnp.concatenate([gW, zpad_w, cW], axis=1)
    b_comb = jnp.concatenate(
        [gb, jnp.zeros((PADW - HIDDEN,), jnp.float32), cb]).reshape(1, WCOMB)

    x_out, log_out = pl.pallas_call(
        _fused_kernel,
        grid=(NB,),
        in_specs=[
            pl.BlockSpec((1, 1, BLK), lambda i: (i, 0, 0)),
            pl.BlockSpec((1, 1, BLK), lambda i: (i, 0, 0)),
            pl.BlockSpec((NUM_ENTS, HIDDEN), lambda i: (0, 0)),
            pl.BlockSpec((NUM_RELS, HIDDEN), lambda i: (0, 0)),
            pl.BlockSpec((1, HIDDEN), lambda i: (0, 0)),
            pl.BlockSpec((3 * HIDDEN, WCOMB), lambda i: (0, 0)),
            pl.BlockSpec((1, WCOMB), lambda i: (0, 0)),
        ],
        out_specs=[
            pl.BlockSpec((BLK, 3 * HIDDEN), lambda i: (i, 0)),
            pl.BlockSpec((BLK, HIDDEN), lambda i: (i, 0)),
        ],
        out_shape=[
            jax.ShapeDtypeStruct((B, 3 * HIDDEN), jnp.float32),
            jax.ShapeDtypeStruct((B, HIDDEN), jnp.float32),
        ],
        compiler_params=pltpu.CompilerParams(
            dimension_semantics=("parallel",)),
    )(sub3, rel3, ent_emb, rel_emb, tim_emb, W_comb, b_comb)
    return log_out, x_out


# speedup vs baseline: 1.1512x; 1.0365x over previous
import jax
import jax.numpy as jnp
from jax.experimental import pallas as pl
from jax.experimental.pallas import tpu as pltpu

NUM_ENTS = 1000
NUM_RELS = 24
HIDDEN = 200
B = 8192
BLK = 2048
HALF = BLK // 4
NB = B // BLK
PADW = 256
WCOMB = PADW + HIDDEN
LOG_HALF = -0.6931471805599453


def _fused_kernel(sub_ref, rel_ref, ent_ref, rele_ref, tim_ref,
                  W_ref, b_ref, x_ref, out_ref):
    sub_row_full = sub_ref[0].astype(jnp.int16)
    rel_row_full = rel_ref[0].astype(jnp.int16)

    one = jnp.ones((), jnp.bfloat16)
    zero = jnp.zeros((), jnp.bfloat16)
    ent_b = ent_ref[...].astype(jnp.bfloat16)
    rele_b = rele_ref[...].astype(jnp.bfloat16)
    tim = tim_ref[...]
    W = W_ref[...].astype(jnp.bfloat16)
    tim_score = (jnp.dot(tim.astype(jnp.bfloat16), W[2 * HIDDEN:],
                         preferred_element_type=jnp.float32)
                 + b_ref[...])
    ent_iota = jax.lax.broadcasted_iota(jnp.int16, (NUM_ENTS, HALF), 0)
    rel_iota = jax.lax.broadcasted_iota(jnp.int16, (NUM_RELS, HALF), 0)

    for h in range(4):
        lo = h * HALF
        sub_row = sub_row_full[:, lo:lo + HALF]
        rel_row = rel_row_full[:, lo:lo + HALF]

        sub_oh = jnp.where(ent_iota == sub_row, one, zero)
        sub_e = jax.lax.dot_general(
            sub_oh, ent_b,
            dimension_numbers=(((0,), (0,)), ((), ())),
            preferred_element_type=jnp.float32)

        rel_oh = jnp.where(rel_iota == rel_row, one, zero)
        rel_e = jax.lax.dot_general(
            rel_oh, rele_b,
            dimension_numbers=(((0,), (0,)), ((), ())),
            preferred_element_type=jnp.float32)

        rows = pl.ds(lo, HALF)
        x_ref[rows, pl.ds(0, HIDDEN)] = sub_e
        x_ref[rows, pl.ds(HIDDEN, HIDDEN)] = rel_e
        x_ref[rows, pl.ds(2 * HIDDEN, HIDDEN)] = jnp.broadcast_to(
            tim, (HALF, HIDDEN))

        scores = (
            jax.lax.dot_general(sub_e.astype(jnp.bfloat16), W[:HIDDEN],
                                dimension_numbers=(((1,), (0,)), ((), ())),
                                preferred_element_type=jnp.float32)
            + jax.lax.dot_general(rel_e.astype(jnp.bfloat16),
                                  W[HIDDEN:2 * HIDDEN],
                                  dimension_numbers=(((1,), (0,)), ((), ())),
                                  preferred_element_type=jnp.float32)
            + tim_score)

        gscore = scores[:, :HIDDEN]
        z = jnp.tanh(scores[:, PADW:PADW + HIDDEN])
        ez = jnp.exp(z)
        cscore = ez / jnp.sum(ez, axis=1, keepdims=True)
        out_ref[rows, :] = jnp.log(gscore + cscore) + jnp.float32(LOG_HALF)


def kernel(hist_sub, hist_rel, hist_obj, sub, rel, obj,
           ent_emb, rel_emb, tim_emb, gW, gb, cW, cb):
    del hist_sub, hist_rel, hist_obj, obj
    sub3 = sub.astype(jnp.int32).reshape(NB, 1, BLK)
    rel3 = rel.astype(jnp.int32).reshape(NB, 1, BLK)
    zpad_w = jnp.zeros((3 * HIDDEN, PADW - HIDDEN), jnp.float32)
    W_comb = jnp.concatenate([gW, zpad_w, cW], axis=1)
    b_comb = jnp.concatenate(
        [gb, jnp.zeros((PADW - HIDDEN,), jnp.float32), cb]).reshape(1, WCOMB)

    x_out, log_out = pl.pallas_call(
        _fused_kernel,
        grid=(NB,),
        in_specs=[
            pl.BlockSpec((1, 1, BLK), lambda i: (i, 0, 0)),
            pl.BlockSpec((1, 1, BLK), lambda i: (i, 0, 0)),
            pl.BlockSpec((NUM_ENTS, HIDDEN), lambda i: (0, 0)),
            pl.BlockSpec((NUM_RELS, HIDDEN), lambda i: (0, 0)),
            pl.BlockSpec((1, HIDDEN), lambda i: (0, 0)),
            pl.BlockSpec((3 * HIDDEN, WCOMB), lambda i: (0, 0)),
            pl.BlockSpec((1, WCOMB), lambda i: (0, 0)),
        ],
        out_specs=[
            pl.BlockSpec((BLK, 3 * HIDDEN), lambda i: (i, 0)),
            pl.BlockSpec((BLK, HIDDEN), lambda i: (i, 0)),
        ],
        out_shape=[
            jax.ShapeDtypeStruct((B, 3 * HIDDEN), jnp.float32),
            jax.ShapeDtypeStruct((B, HIDDEN), jnp.float32),
        ],
        compiler_params=pltpu.CompilerParams(
            dimension_semantics=("parallel",)),
    )(sub3, rel3, ent_emb, rel_emb, tim_emb, W_comb, b_comb)
    return log_out, x_out
